# baseline (speedup 1.0000x reference)
import jax
import jax.numpy as jnp
from jax.experimental import pallas as pl
from jax.experimental.pallas import tpu as pltpu

B = 1024
L = 1024
VOCAB = 21
SEQ_EMB = 44
POS_EMB = 64
PSSM_D = POS_EMB - SEQ_EMB
G = 32
RG = L // G
NV = G // 2
BB = 16


def _tc_kernel(c_ref, pssm_ref, m_ref, pos_ref, out_ref):
    lane = jax.lax.broadcasted_iota(jnp.int32, (1, 1, 128), 2)
    target = jnp.where(lane < VOCAB, lane,
                       jnp.where(lane < 2 * VOCAB, lane - VOCAB, 31))
    m = m_ref[...]
    c16 = c_ref[...]
    p = pssm_ref[...]
    pos = pos_ref[...]

    for k in range(NV):
        ck = jnp.broadcast_to(c16[:, :, k:k + 1], (BB, RG, 128))
        val = jnp.where(lane < VOCAB, ck & 0xFFFF, ck >> 16)
        oh = (val == target).astype(jnp.float32)[:, :, :2 * VOCAB]
        emb = jax.lax.dot_general(
            oh.reshape(BB * RG, 2 * VOCAB), m,
            (((1,), (0,)), ((), ())),
            preferred_element_type=jnp.float32,
        ).reshape(BB, RG, 128)
        pk = p[:, :, 2 * PSSM_D * k:2 * PSSM_D * (k + 1)]
        pk = jnp.pad(pk, ((0, 0), (0, 0), (0, 128 - 2 * PSSM_D)))
        ps = jnp.where((lane >= SEQ_EMB) & (lane < POS_EMB),
                       jnp.roll(pk, SEQ_EMB, axis=-1), 0.0)
        ps = ps + jnp.where(lane >= POS_EMB + SEQ_EMB,
                            jnp.roll(pk, 2 * SEQ_EMB, axis=-1), 0.0)
        out_ref[:, :, 128 * k:128 * (k + 1)] = (
            emb + ps + pos[None, :, 128 * k:128 * (k + 1)])


def kernel(seq, pssm, token_table, pos_table):
    seq16 = seq.astype(jnp.int16).reshape(B, L // 2, 2)
    c = jax.lax.bitcast_convert_type(seq16, jnp.int32).reshape(B, RG, NV)
    pssm_g = pssm.reshape(B, RG, G * PSSM_D)
    pos_g = pos_table.reshape(RG, G * POS_EMB)
    m = jnp.zeros((2 * VOCAB, 128), jnp.float32)
    m = m.at[:VOCAB, :SEQ_EMB].set(token_table)
    m = m.at[VOCAB:, POS_EMB:POS_EMB + SEQ_EMB].set(token_table)

    grid = (B // BB,)
    out = pl.pallas_call(
        _tc_kernel,
        grid=grid,
        in_specs=[
            pl.BlockSpec((BB, RG, NV), lambda i: (i, 0, 0)),
            pl.BlockSpec((BB, RG, G * PSSM_D), lambda i: (i, 0, 0)),
            pl.BlockSpec((2 * VOCAB, 128), lambda i: (0, 0)),
            pl.BlockSpec((RG, G * POS_EMB), lambda i: (0, 0)),
        ],
        out_specs=pl.BlockSpec((BB, RG, G * POS_EMB), lambda i: (i, 0, 0)),
        out_shape=jax.ShapeDtypeStruct((B, RG, G * POS_EMB), jnp.float32),
        compiler_params=pltpu.CompilerParams(
            dimension_semantics=("arbitrary",),
        ),
    )(c, pssm_g, m, pos_g)
    return out.reshape(B, L, POS_EMB)


# speedup vs baseline: 1.1594x; 1.1594x over previous
import jax
import jax.numpy as jnp
from jax.experimental import pallas as pl
from jax.experimental.pallas import tpu as pltpu

B = 1024
L = 1024
VOCAB = 21
SEQ_EMB = 44
POS_EMB = 64
PSSM_D = POS_EMB - SEQ_EMB
G = 32
RG = L // G
NV = G // 2
BB = 32


def _tc_kernel(c_ref, pssm_ref, m_ref, pos_ref, out_ref):
    lane = jax.lax.broadcasted_iota(jnp.int32, (1, 1, 128), 2)
    target = jnp.where(lane < VOCAB, lane,
                       jnp.where(lane < 2 * VOCAB, lane - VOCAB, 31))
    m = m_ref[...]
    c16 = c_ref[...]
    p = pssm_ref[...]
    pos = pos_ref[...]

    for k in range(NV):
        ck = jnp.broadcast_to(c16[:, :, k:k + 1], (BB, RG, 128))
        val = jnp.where(lane < VOCAB, ck & 0xFFFF, ck >> 16)
        oh = (val == target).astype(jnp.bfloat16)
        pk = p[:, :, 2 * PSSM_D * k:2 * PSSM_D * (k + 1)]
        pk = jnp.pad(pk, ((0, 0), (0, 0), (0, 128 - 2 * PSSM_D)))
        x = jnp.where(lane < 2 * VOCAB, oh,
                      jnp.roll(pk, 2 * VOCAB, axis=-1).astype(jnp.bfloat16))
        emb = jax.lax.dot_general(
            x.reshape(BB * RG, 128), m,
            (((1,), (0,)), ((), ())),
            preferred_element_type=jnp.float32,
        ).reshape(BB, RG, 128)
        out_ref[:, :, 128 * k:128 * (k + 1)] = (
            emb + pos[None, :, 128 * k:128 * (k + 1)])


def kernel(seq, pssm, token_table, pos_table):
    seq16 = seq.astype(jnp.int16).reshape(B, L // 2, 2)
    c = jax.lax.bitcast_convert_type(seq16, jnp.int32).reshape(B, RG, NV)
    pssm_g = pssm.reshape(B, RG, G * PSSM_D)
    pos_g = pos_table.reshape(RG, G * POS_EMB)
    m = jnp.zeros((128, 128), jnp.float32)
    m = m.at[:VOCAB, :SEQ_EMB].set(token_table)
    m = m.at[VOCAB:2 * VOCAB, POS_EMB:POS_EMB + SEQ_EMB].set(token_table)
    eye = jnp.eye(PSSM_D, dtype=jnp.float32)
    m = m.at[2 * VOCAB:2 * VOCAB + PSSM_D, SEQ_EMB:POS_EMB].set(eye)
    m = m.at[2 * VOCAB + PSSM_D:2 * (VOCAB + PSSM_D),
             POS_EMB + SEQ_EMB:2 * POS_EMB].set(eye)
    m = m.astype(jnp.bfloat16)

    grid = (B // BB,)
    out = pl.pallas_call(
        _tc_kernel,
        grid=grid,
        in_specs=[
            pl.BlockSpec((BB, RG, NV), lambda i: (i, 0, 0)),
            pl.BlockSpec((BB, RG, G * PSSM_D), lambda i: (i, 0, 0)),
            pl.BlockSpec((128, 128), lambda i: (0, 0)),
            pl.BlockSpec((RG, G * POS_EMB), lambda i: (0, 0)),
        ],
        out_specs=pl.BlockSpec((BB, RG, G * POS_EMB), lambda i: (i, 0, 0)),
        out_shape=jax.ShapeDtypeStruct((B, RG, G * POS_EMB), jnp.float32),
        compiler_params=pltpu.CompilerParams(
            dimension_semantics=("arbitrary",),
        ),
    )(c, pssm_g, m, pos_g)
    return out.reshape(B, L, POS_EMB)


# speedup vs baseline: 1.1738x; 1.0124x over previous
import jax
import jax.numpy as jnp
from jax.experimental import pallas as pl
from jax.experimental.pallas import tpu as pltpu

B = 1024
L = 1024
VOCAB = 21
SEQ_EMB = 44
POS_EMB = 64
PSSM_D = POS_EMB - SEQ_EMB
G = 32
RG = L // G
NV = G // 2
BB = 32


def _tc_kernel(c_ref, pssm_ref, m_ref, pos_ref, out_ref):
    lane = jax.lax.broadcasted_iota(jnp.int32, (1, 1, 128), 2)
    target = jnp.where(lane < VOCAB, lane,
                       jnp.where(lane < 2 * VOCAB, lane - VOCAB, 31))
    m = m_ref[...]
    c16 = c_ref[...]
    p = pssm_ref[...]
    pos = pos_ref[...]

    for k in range(NV):
        ck = jnp.broadcast_to(c16[:, :, k:k + 1], (BB, RG, 128))
        val = jnp.where(lane < VOCAB, ck & 0xFFFF, ck >> 16)
        oh = (val == target).astype(jnp.float32)
        pk = p[:, :, 2 * PSSM_D * k:2 * PSSM_D * (k + 1)]
        pk = jnp.pad(pk, ((0, 0), (0, 0), (0, 128 - 2 * PSSM_D)))
        x = jnp.where(lane < 2 * VOCAB, oh,
                      jnp.roll(pk, 2 * VOCAB, axis=-1))
        emb = jax.lax.dot_general(
            x.reshape(BB * RG, 128), m,
            (((1,), (0,)), ((), ())),
            preferred_element_type=jnp.float32,
        ).reshape(BB, RG, 128)
        out_ref[:, :, 128 * k:128 * (k + 1)] = (
            emb + pos[None, :, 128 * k:128 * (k + 1)])


def kernel(seq, pssm, token_table, pos_table):
    seq16 = seq.astype(jnp.int16).reshape(B, L // 2, 2)
    c = jax.lax.bitcast_convert_type(seq16, jnp.int32).reshape(B, RG, NV)
    pssm_g = pssm.reshape(B, RG, G * PSSM_D)
    pos_g = pos_table.reshape(RG, G * POS_EMB)
    m = jnp.zeros((128, 128), jnp.float32)
    m = m.at[:VOCAB, :SEQ_EMB].set(token_table)
    m = m.at[VOCAB:2 * VOCAB, POS_EMB:POS_EMB + SEQ_EMB].set(token_table)
    eye = jnp.eye(PSSM_D, dtype=jnp.float32)
    m = m.at[2 * VOCAB:2 * VOCAB + PSSM_D, SEQ_EMB:POS_EMB].set(eye)
    m = m.at[2 * VOCAB + PSSM_D:2 * (VOCAB + PSSM_D),
             POS_EMB + SEQ_EMB:2 * POS_EMB].set(eye)

    grid = (B // BB,)
    out = pl.pallas_call(
        _tc_kernel,
        grid=grid,
        in_specs=[
            pl.BlockSpec((BB, RG, NV), lambda i: (i, 0, 0)),
            pl.BlockSpec((BB, RG, G * PSSM_D), lambda i: (i, 0, 0)),
            pl.BlockSpec((128, 128), lambda i: (0, 0)),
            pl.BlockSpec((RG, G * POS_EMB), lambda i: (0, 0)),
        ],
        out_specs=pl.BlockSpec((BB, RG, G * POS_EMB), lambda i: (i, 0, 0)),
        out_shape=jax.ShapeDtypeStruct((B, RG, G * POS_EMB), jnp.float32),
        compiler_params=pltpu.CompilerParams(
            dimension_semantics=("arbitrary",),
        ),
    )(c, pssm_g, m, pos_g)
    return out.reshape(B, L, POS_EMB)
